# baseline (speedup 1.0000x reference)
import jax
import jax.numpy as jnp
from jax.experimental import pallas as pl
from jax.experimental.pallas import tpu as pltpu

B, A, LIDAR = 256, 64, 20
H, OUT, L = 128, 8, 2
FEAT = 18 + LIDAR
E = 64
R = E * A


def _relu(x):
    return jnp.maximum(x, 0.0)


def _ln(x, g=None, b=None):
    m = jnp.mean(x, axis=-1, keepdims=True)
    xc = x - m
    v = jnp.mean(xc * xc, axis=-1, keepdims=True)
    y = xc * jax.lax.rsqrt(v + 1e-5)
    if g is not None:
        y = y * g + b
    return y


def _dot(a, b):
    return jnp.dot(a, b, preferred_element_type=jnp.float32)


def _fused(data_ref, lW1_ref, lb1_ref, lW2_ref, lb2_ref,
           g1W1a_ref, g1W1b_ref, g1b1_ref, g1W2_ref, g1b2_ref,
           g2W1_ref, g2b1_ref, g2W2_ref, g2b2_ref,
           tWv_ref, tbv_ref, tWo_ref, tbo_ref,
           tF1_ref, tf1b_ref, tF2_ref, tf2b_ref,
           ln1g_ref, ln1b_ref, ln2g_ref, ln2b_ref, ln2gT_ref,
           fcW_ref, fcb_ref, out_ref):
    d = data_ref[...]
    orig = d[:, :18]
    lid = d[:, 18:FEAT]

    hl = _relu(_dot(lid, lW1_ref[...]) + lb1_ref[...])
    lidf = _relu(_dot(hl, lW2_ref[...]) + lb2_ref[...])

    p3 = d[:, :2].reshape(E, A, 2)
    g3 = jax.lax.dot_general(p3, p3, (((2,), (2,)), ((0,), (0,))),
                             preferred_element_type=jnp.float32)
    ii = jax.lax.broadcasted_iota(jnp.int32, (E, A, A), 1)
    jj = jax.lax.broadcasted_iota(jnp.int32, (E, A, A), 2)
    gd = jnp.where(ii == jj, g3, 0.0)
    diag_c = jnp.sum(gd, axis=2, keepdims=True)
    diag_r = jnp.sum(gd, axis=1, keepdims=True)
    dist2 = diag_c + diag_r - 2.0 * g3
    mp3 = jnp.where(dist2 <= 100.0, 1.0, 0.0)

    def _agg(t):
        t3 = t.reshape(E, A, H)
        a3 = jax.lax.dot_general(mp3, t3, (((2,), (1,)), ((0,), (0,))),
                                 preferred_element_type=jnp.float32)
        return a3.reshape(R, H)

    t1 = _dot(orig, g1W1a_ref[...]) + _dot(lidf, g1W1b_ref[...])
    h1 = _relu(_agg(t1) + g1b1_ref[...])
    x = _relu(_dot(h1, g1W2_ref[...]) + g1b2_ref[...])

    t2 = _dot(x, g2W1_ref[...])
    h2 = _relu(_agg(t2) + g2b1_ref[...])
    x = _relu(_dot(h2, g2W2_ref[...]) + g2b2_ref[...])

    for l in range(L):
        r0, r1 = l * H, (l + 1) * H
        Wvo = _dot(tWv_ref[r0:r1, :], tWo_ref[r0:r1, :])
        bvo = _dot(tbv_ref[l:l + 1, :], tWo_ref[r0:r1, :]) + tbo_ref[l:l + 1, :]
        o = _dot(x, Wvo) + bvo
        x = _ln(x + o, ln1g_ref[l:l + 1, :], ln1b_ref[l:l + 1, :])
        ff = _dot(_relu(_dot(x, tF1_ref[r0:r1, :]) + tf1b_ref[l:l + 1, :]),
                  tF2_ref[r0:r1, :]) + tf2b_ref[l:l + 1, :]
        if l < L - 1:
            x = _ln(x + ff, ln2g_ref[l:l + 1, :], ln2b_ref[l:l + 1, :])
        else:
            u = _ln(x + ff)
            fcWg = ln2gT_ref[...] * fcW_ref[...]
            fcbg = _dot(ln2b_ref[l:l + 1, :], fcW_ref[...]) + fcb_ref[...]
            out_ref[...] = _dot(u, fcWg) + fcbg
            return


def _full(shape):
    nd = len(shape)
    return pl.BlockSpec(shape, lambda i: (0,) * nd)


def kernel(data, lW1, lb1, lW2, lb2, g1W1, g1b1, g1W2, g1b2,
           g2W1, g2b1, g2W2, g2b2, tWq, tbq, tWk, tbk, tWv, tbv,
           tWo, tbo, tF1, tf1b, tF2, tf2b, ln1g, ln1b, ln2g, ln2b,
           fcW, fcb):
    del tWq, tbq, tWk, tbk
    data2 = data.reshape(B * A, FEAT)
    ops = (
        data2,
        lW1, lb1.reshape(1, H), lW2, lb2.reshape(1, LIDAR),
        g1W1[:18], g1W1[18:], g1b1.reshape(1, H), g1W2, g1b2.reshape(1, H),
        g2W1, g2b1.reshape(1, H), g2W2, g2b2.reshape(1, H),
        tWv.reshape(L * H, H), tbv, tWo.reshape(L * H, H), tbo,
        tF1.reshape(L * H, H), tf1b, tF2.reshape(L * H, H), tf2b,
        ln1g, ln1b, ln2g, ln2b, ln2g[L - 1].reshape(H, 1),
        fcW, fcb.reshape(1, OUT),
    )
    in_specs = [pl.BlockSpec((R, FEAT), lambda i: (i, 0))]
    in_specs += [_full(o.shape) for o in ops[1:]]
    out = pl.pallas_call(
        _fused,
        grid=(B * A // R,),
        in_specs=in_specs,
        out_specs=pl.BlockSpec((R, OUT), lambda i: (i, 0)),
        out_shape=jax.ShapeDtypeStruct((B * A, OUT), jnp.float32),
        compiler_params=pltpu.CompilerParams(
            dimension_semantics=("parallel",)),
    )(*ops)
    return out.reshape(B, A, OUT)


# speedup vs baseline: 1.0773x; 1.0773x over previous
import jax
import jax.numpy as jnp
from jax.experimental import pallas as pl
from jax.experimental.pallas import tpu as pltpu

B, A, LIDAR = 256, 64, 20
H, OUT, L = 128, 8, 2
FEAT = 18 + LIDAR
E = 64
R = E * A


def _relu(x):
    return jnp.maximum(x, 0.0)


def _ln(x, g=None, b=None):
    m = jnp.mean(x, axis=-1, keepdims=True)
    xc = x - m
    v = jnp.mean(xc * xc, axis=-1, keepdims=True)
    y = xc * jax.lax.rsqrt(v + 1e-5)
    if g is not None:
        y = y * g + b
    return y


def _dot(a, b):
    return jnp.dot(a, b, preferred_element_type=jnp.float32)


def _fused(data_ref, lW1_ref, lb1_ref, lW2_ref, lb2_ref,
           g1W1_ref, g1b1_ref, g1W2_ref, g1b2_ref,
           g2W1_ref, g2b1_ref, g2W2_ref, g2b2_ref,
           tWv_ref, tbv_ref, tWo_ref, tbo_ref,
           tF1_ref, tf1b_ref, tF2_ref, tf2b_ref,
           ln1g_ref, ln1b_ref, ln2g_ref, ln2b_ref,
           fcW_ref, fcb_ref, out_ref):
    d = data_ref[...].reshape(R, FEAT)
    orig = d[:, :18]
    lid = d[:, 18:FEAT]

    hl = _relu(_dot(lid, lW1_ref[...]) + lb1_ref[...].reshape(1, H))
    lidf = _relu(_dot(hl, lW2_ref[...]) + lb2_ref[...].reshape(1, LIDAR))

    p3 = d[:, :2].reshape(E, A, 2)
    g3 = jax.lax.dot_general(p3, p3, (((2,), (2,)), ((0,), (0,))),
                             preferred_element_type=jnp.float32)
    ii = jax.lax.broadcasted_iota(jnp.int32, (E, A, A), 1)
    jj = jax.lax.broadcasted_iota(jnp.int32, (E, A, A), 2)
    gd = jnp.where(ii == jj, g3, 0.0)
    diag_c = jnp.sum(gd, axis=2, keepdims=True)
    diag_r = jnp.sum(gd, axis=1, keepdims=True)
    dist2 = diag_c + diag_r - 2.0 * g3
    mp3 = jnp.where(dist2 <= 100.0, 1.0, 0.0)

    def _agg(t):
        t3 = t.reshape(E, A, H)
        a3 = jax.lax.dot_general(mp3, t3, (((2,), (1,)), ((0,), (0,))),
                                 preferred_element_type=jnp.float32)
        return a3.reshape(R, H)

    t1 = _dot(orig, g1W1_ref[0:18, :]) + _dot(lidf, g1W1_ref[18:FEAT, :])
    h1 = _relu(_agg(t1) + g1b1_ref[...].reshape(1, H))
    x = _relu(_dot(h1, g1W2_ref[...]) + g1b2_ref[...].reshape(1, H))

    t2 = _dot(x, g2W1_ref[...])
    h2 = _relu(_agg(t2) + g2b1_ref[...].reshape(1, H))
    x = _relu(_dot(h2, g2W2_ref[...]) + g2b2_ref[...].reshape(1, H))

    for l in range(L):
        Wo = tWo_ref[l]
        Wvo = _dot(tWv_ref[l], Wo)
        bvo = _dot(tbv_ref[l:l + 1, :], Wo) + tbo_ref[l:l + 1, :]
        o = _dot(x, Wvo) + bvo
        x = _ln(x + o, ln1g_ref[l:l + 1, :], ln1b_ref[l:l + 1, :])
        ff = _dot(_relu(_dot(x, tF1_ref[l]) + tf1b_ref[l:l + 1, :]),
                  tF2_ref[l]) + tf2b_ref[l:l + 1, :]
        x = _ln(x + ff, ln2g_ref[l:l + 1, :], ln2b_ref[l:l + 1, :])

    out = _dot(x, fcW_ref[...]) + fcb_ref[...].reshape(1, OUT)
    out_ref[...] = out.reshape(E, A, OUT)


def _full(shape):
    nd = len(shape)
    return pl.BlockSpec(shape, lambda i: (0,) * nd)


def kernel(data, lW1, lb1, lW2, lb2, g1W1, g1b1, g1W2, g1b2,
           g2W1, g2b1, g2W2, g2b2, tWq, tbq, tWk, tbk, tWv, tbv,
           tWo, tbo, tF1, tf1b, tF2, tf2b, ln1g, ln1b, ln2g, ln2b,
           fcW, fcb):
    del tWq, tbq, tWk, tbk
    ops = (
        data,
        lW1, lb1, lW2, lb2,
        g1W1, g1b1, g1W2, g1b2,
        g2W1, g2b1, g2W2, g2b2,
        tWv, tbv, tWo, tbo,
        tF1, tf1b, tF2, tf2b,
        ln1g, ln1b, ln2g, ln2b,
        fcW, fcb,
    )
    in_specs = [pl.BlockSpec((E, A, FEAT), lambda i: (i, 0, 0))]
    in_specs += [_full(o.shape) for o in ops[1:]]
    return pl.pallas_call(
        _fused,
        grid=(B // E,),
        in_specs=in_specs,
        out_specs=pl.BlockSpec((E, A, OUT), lambda i: (i, 0, 0)),
        out_shape=jax.ShapeDtypeStruct((B, A, OUT), jnp.float32),
        compiler_params=pltpu.CompilerParams(
            dimension_semantics=("parallel",)),
    )(*ops)


# speedup vs baseline: 1.0781x; 1.0008x over previous
import jax
import jax.numpy as jnp
from jax.experimental import pallas as pl
from jax.experimental.pallas import tpu as pltpu

B, A, LIDAR = 256, 64, 20
H, OUT, L = 128, 8, 2
FEAT = 18 + LIDAR
E = 128
R = E * A


def _relu(x):
    return jnp.maximum(x, 0.0)


def _ln(x, g=None, b=None):
    m = jnp.mean(x, axis=-1, keepdims=True)
    xc = x - m
    v = jnp.mean(xc * xc, axis=-1, keepdims=True)
    y = xc * jax.lax.rsqrt(v + 1e-5)
    if g is not None:
        y = y * g + b
    return y


def _dot(a, b):
    return jnp.dot(a, b, preferred_element_type=jnp.float32)


def _fused(data_ref, lW1_ref, lb1_ref, lW2_ref, lb2_ref,
           g1W1_ref, g1b1_ref, g1W2_ref, g1b2_ref,
           g2W1_ref, g2b1_ref, g2W2_ref, g2b2_ref,
           tWv_ref, tbv_ref, tWo_ref, tbo_ref,
           tF1_ref, tf1b_ref, tF2_ref, tf2b_ref,
           ln1g_ref, ln1b_ref, ln2g_ref, ln2b_ref,
           fcW_ref, fcb_ref, out_ref):
    d = data_ref[...].reshape(R, FEAT)
    orig = d[:, :18]
    lid = d[:, 18:FEAT]

    hl = _relu(_dot(lid, lW1_ref[...]) + lb1_ref[...].reshape(1, H))
    lidf = _relu(_dot(hl, lW2_ref[...]) + lb2_ref[...].reshape(1, LIDAR))

    p3 = d[:, :2].reshape(E, A, 2)
    g3 = jax.lax.dot_general(p3, p3, (((2,), (2,)), ((0,), (0,))),
                             preferred_element_type=jnp.float32)
    ii = jax.lax.broadcasted_iota(jnp.int32, (E, A, A), 1)
    jj = jax.lax.broadcasted_iota(jnp.int32, (E, A, A), 2)
    gd = jnp.where(ii == jj, g3, 0.0)
    diag_c = jnp.sum(gd, axis=2, keepdims=True)
    diag_r = jnp.sum(gd, axis=1, keepdims=True)
    dist2 = diag_c + diag_r - 2.0 * g3
    mp3 = jnp.where(dist2 <= 100.0, 1.0, 0.0)

    def _agg(t):
        t3 = t.reshape(E, A, H)
        a3 = jax.lax.dot_general(mp3, t3, (((2,), (1,)), ((0,), (0,))),
                                 preferred_element_type=jnp.float32)
        return a3.reshape(R, H)

    t1 = _dot(orig, g1W1_ref[0:18, :]) + _dot(lidf, g1W1_ref[18:FEAT, :])
    h1 = _relu(_agg(t1) + g1b1_ref[...].reshape(1, H))
    x = _relu(_dot(h1, g1W2_ref[...]) + g1b2_ref[...].reshape(1, H))

    t2 = _dot(x, g2W1_ref[...])
    h2 = _relu(_agg(t2) + g2b1_ref[...].reshape(1, H))
    x = _relu(_dot(h2, g2W2_ref[...]) + g2b2_ref[...].reshape(1, H))

    for l in range(L):
        Wo = tWo_ref[l]
        Wvo = _dot(tWv_ref[l], Wo)
        bvo = _dot(tbv_ref[l:l + 1, :], Wo) + tbo_ref[l:l + 1, :]
        o = _dot(x, Wvo) + bvo
        x = _ln(x + o, ln1g_ref[l:l + 1, :], ln1b_ref[l:l + 1, :])
        ff = _dot(_relu(_dot(x, tF1_ref[l]) + tf1b_ref[l:l + 1, :]),
                  tF2_ref[l]) + tf2b_ref[l:l + 1, :]
        x = _ln(x + ff, ln2g_ref[l:l + 1, :], ln2b_ref[l:l + 1, :])

    out = _dot(x, fcW_ref[...]) + fcb_ref[...].reshape(1, OUT)
    out_ref[...] = out.reshape(E, A, OUT)


def _full(shape):
    nd = len(shape)
    return pl.BlockSpec(shape, lambda i: (0,) * nd)


def kernel(data, lW1, lb1, lW2, lb2, g1W1, g1b1, g1W2, g1b2,
           g2W1, g2b1, g2W2, g2b2, tWq, tbq, tWk, tbk, tWv, tbv,
           tWo, tbo, tF1, tf1b, tF2, tf2b, ln1g, ln1b, ln2g, ln2b,
           fcW, fcb):
    del tWq, tbq, tWk, tbk
    ops = (
        data,
        lW1, lb1, lW2, lb2,
        g1W1, g1b1, g1W2, g1b2,
        g2W1, g2b1, g2W2, g2b2,
        tWv, tbv, tWo, tbo,
        tF1, tf1b, tF2, tf2b,
        ln1g, ln1b, ln2g, ln2b,
        fcW, fcb,
    )
    in_specs = [pl.BlockSpec((E, A, FEAT), lambda i: (i, 0, 0))]
    in_specs += [_full(o.shape) for o in ops[1:]]
    return pl.pallas_call(
        _fused,
        grid=(B // E,),
        in_specs=in_specs,
        out_specs=pl.BlockSpec((E, A, OUT), lambda i: (i, 0, 0)),
        out_shape=jax.ShapeDtypeStruct((B, A, OUT), jnp.float32),
        compiler_params=pltpu.CompilerParams(
            dimension_semantics=("parallel",)),
    )(*ops)
